# baseline (speedup 1.0000x reference)
import functools
import math

import jax
import jax.numpy as jnp
from jax.experimental import pallas as pl
from jax.experimental.pallas import tpu as pltpu

B, LQ, LK, D = 4, 2048, 2048, 2048
DP = 2176

F32 = jnp.float32


def _maug_body(lhs_ref, wk_ref, o_ref):
    o_ref[...] = jax.lax.dot_general(
        lhs_ref[...], wk_ref[...], (((1,), (0,)), ((), ())),
        preferred_element_type=F32)


def _maug(lhsT, wk):
    bm, bn = 1088, 1024
    return pl.pallas_call(
        _maug_body,
        grid=(DP // bm, D // bn),
        in_specs=[
            pl.BlockSpec((bm, D), lambda i, j: (i, 0)),
            pl.BlockSpec((D, bn), lambda i, j: (0, j)),
        ],
        out_specs=pl.BlockSpec((bm, bn), lambda i, j: (i, j)),
        out_shape=jax.ShapeDtypeStruct((DP, D), F32),
        compiler_params=pltpu.CompilerParams(
            dimension_semantics=("parallel", "arbitrary"),
            vmem_limit_bytes=100 * 1024 * 1024,
        ),
    )(lhsT, wk)


def _p_body(x_ref, w_ref, o_ref):
    o_ref[...] = jax.lax.dot_general(
        x_ref[...], w_ref[...], (((1,), (0,)), ((), ())),
        preferred_element_type=F32)


def _pmat(q1, maug):
    bm, bn = 1024, 1024
    m = B * LQ
    return pl.pallas_call(
        _p_body,
        grid=(m // bm, D // bn),
        in_specs=[
            pl.BlockSpec((bm, DP), lambda i, j: (i, 0)),
            pl.BlockSpec((DP, bn), lambda i, j: (0, j)),
        ],
        out_specs=pl.BlockSpec((bm, bn), lambda i, j: (i, j)),
        out_shape=jax.ShapeDtypeStruct((m, D), F32),
        compiler_params=pltpu.CompilerParams(
            dimension_semantics=("parallel", "arbitrary"),
            vmem_limit_bytes=100 * 1024 * 1024,
        ),
    )(q1, maug)


TQ = 1024
TK = 256
NQ = LQ // TQ
NK = LK // TK


def _attn_body(p_ref, k_ref, v_ref, m_ref, o_ref, s_ref):
    j = pl.program_id(1)
    pt = p_ref[0]
    kt = k_ref[0]
    vt = v_ref[0]
    mt = m_ref[...]

    s = jax.lax.dot_general(pt, kt, (((1,), (1,)), ((), ())),
                            preferred_element_type=F32)
    s_ref[j] = s + mt

    mx = jnp.dot(mt, vt, preferred_element_type=F32)

    @pl.when(j == 0)
    def _():
        o_ref[0] = mx

    @pl.when(j != 0)
    def _():
        o_ref[0] = o_ref[0] + mx

    @pl.when(j == NK - 1)
    def _():
        m = jnp.max(s_ref[0], axis=-1, keepdims=True)
        for t in range(1, NK):
            m = jnp.maximum(m, jnp.max(s_ref[t], axis=-1, keepdims=True))
        den = jnp.zeros_like(m)
        for t in range(NK):
            e = jnp.exp(s_ref[t] - m)
            s_ref[t] = e
            den = den + jnp.sum(e, axis=-1, keepdims=True)
        r = 1.0 / den
        for t in range(NK):
            sl = slice(t * TK, (t + 1) * TK)
            o_ref[0, :, sl] = o_ref[0, :, sl] * (s_ref[t] * r)


def _attn(p, key, value, mask):
    g = B * NQ
    return pl.pallas_call(
        _attn_body,
        grid=(g, NK),
        in_specs=[
            pl.BlockSpec((1, TQ, D), lambda i, j: (i // NQ, i % NQ, 0)),
            pl.BlockSpec((1, TK, D), lambda i, j: (i // NQ, j, 0)),
            pl.BlockSpec((1, TK, D), lambda i, j: (i // NQ, j, 0)),
            pl.BlockSpec((TQ, TK), lambda i, j: (i % NQ, j)),
        ],
        out_specs=pl.BlockSpec((1, TQ, D), lambda i, j: (i // NQ, i % NQ, 0)),
        out_shape=jax.ShapeDtypeStruct((B, LQ, D), F32),
        scratch_shapes=[pltpu.VMEM((NK, TQ, TK), F32)],
        compiler_params=pltpu.CompilerParams(
            dimension_semantics=("parallel", "arbitrary"),
            vmem_limit_bytes=100 * 1024 * 1024,
        ),
    )(p, key, value, mask)


def kernel(query_input, key_input, value_input, Wq, bq, Wk, bk, attn_mask):
    scale = 1.0 / math.sqrt(D)
    lhsT = jnp.concatenate(
        [Wq * scale, (bq * scale)[:, None]], axis=1).T
    lhsT = jnp.pad(lhsT, ((0, DP - (D + 1)), (0, 0)))
    maug = _maug(lhsT, Wk)

    q1 = jnp.concatenate(
        [query_input.reshape(B * LQ, D),
         jnp.ones((B * LQ, 1), F32),
         jnp.zeros((B * LQ, DP - (D + 1)), F32)], axis=1)
    p = _pmat(q1, maug).reshape(B, LQ, D)

    return _attn(p, key_input, value_input, attn_mask)
